# baseline (speedup 1.0000x reference)
import functools

import jax
import jax.numpy as jnp
from jax import lax
from jax.experimental import pallas as pl
from jax.experimental.pallas import tpu as pltpu
from jax.experimental.pallas import tpu_sc as plsc

_NUM_CORES = 2
_NUM_SUBCORES = 16
_NUM_WORKERS = _NUM_CORES * _NUM_SUBCORES
_NBUF = 4


@functools.lru_cache(maxsize=None)
def _make_gather_kernel(num_rows: int, dim: int, chunk: int):
    rows_per_worker = num_rows // _NUM_WORKERS
    num_chunks = rows_per_worker // chunk
    assert num_rows % _NUM_WORKERS == 0
    assert rows_per_worker % chunk == 0
    assert num_chunks % _NBUF == 0 and num_chunks >= 2 * _NBUF
    assert chunk % 8 == 0

    mesh = plsc.VectorSubcoreMesh(
        core_axis_name="c",
        subcore_axis_name="s",
        num_cores=_NUM_CORES,
        num_subcores=_NUM_SUBCORES,
    )

    @functools.partial(
        pl.kernel,
        mesh=mesh,
        out_type=jax.ShapeDtypeStruct((num_rows, dim), jnp.float32),
        scratch_types=[
            pltpu.VMEM((rows_per_worker,), jnp.int32),
            [pltpu.VMEM((chunk, dim), jnp.float32) for _ in range(_NBUF)],
            [pltpu.SemaphoreType.DMA for _ in range(_NBUF)],
            [pltpu.SemaphoreType.DMA for _ in range(_NBUF)],
        ],
    )
    def gather_kernel(table_hbm, idx_hbm, out_hbm, idx_v, bufs, gsems, ssems):
        wid = lax.axis_index("s") * _NUM_CORES + lax.axis_index("c")
        base = wid * rows_per_worker
        pltpu.sync_copy(idx_hbm.at[pl.ds(base, rows_per_worker)], idx_v)

        def start_gather(chunk_id, b):
            off = chunk_id * chunk
            pltpu.async_copy(
                table_hbm.at[idx_v.at[pl.ds(off, chunk)]], bufs[b], gsems[b]
            )

        def wait_gather(b):
            pltpu.make_async_copy(
                table_hbm.at[pl.ds(0, chunk)], bufs[b], gsems[b]
            ).wait()

        def start_store(chunk_id, b):
            pltpu.async_copy(
                bufs[b], out_hbm.at[pl.ds(base + chunk_id * chunk, chunk)], ssems[b]
            )

        def wait_store(b):
            pltpu.make_async_copy(
                bufs[b], out_hbm.at[pl.ds(base, chunk)], ssems[b]
            ).wait()

        start_gather(0, 0)
        start_gather(1, 1)
        wait_gather(0)
        start_store(0, 0)
        start_gather(2, 2)
        wait_gather(1)
        start_store(1, 1)
        start_gather(3, 3)

        def body(g, carry):
            for b in range(_NBUF):
                i = g * _NBUF + b
                wait_store(b)
                start_gather(i, b)
                b2 = (b + 2) % _NBUF
                wait_gather(b2)
                start_store(i - 2, b2)
            return carry

        lax.fori_loop(1, num_chunks // _NBUF, body, 0, unroll=False)

        n = num_chunks
        wait_gather((n - 2) % _NBUF)
        start_store(n - 2, (n - 2) % _NBUF)
        wait_gather((n - 1) % _NBUF)
        start_store(n - 1, (n - 1) % _NBUF)
        for b in range(_NBUF):
            wait_store(b)

    return gather_kernel


def kernel(token_ids, weight):
    dim = weight.shape[1]
    b, s = token_ids.shape
    s_pad = (s + 7) // 8 * 8
    ids = jnp.pad(token_ids.astype(jnp.int32), ((0, 0), (0, s_pad - s)))
    idx = ids.reshape(-1)
    gather = _make_gather_kernel(idx.shape[0], dim, 224)
    out = gather(weight, idx)
    return out.reshape(b, s_pad, dim)[:, :s, :]


# speedup vs baseline: 7.8110x; 7.8110x over previous
import functools

import jax
import jax.numpy as jnp
from jax import lax
from jax.experimental import pallas as pl
from jax.experimental.pallas import tpu as pltpu
from jax.experimental.pallas import tpu_sc as plsc

_NUM_CORES = 2
_NUM_SUBCORES = 16
_NUM_WORKERS = _NUM_CORES * _NUM_SUBCORES
_NBUF = 4


@functools.lru_cache(maxsize=None)
def _make_gather_kernel(batch: int, seq: int, dim: int, seqs_per_chunk: int):
    num_rows = batch * seq
    rows_per_worker = num_rows // _NUM_WORKERS
    seqs_per_worker = batch // _NUM_WORKERS
    chunk = seqs_per_chunk * seq
    num_chunks = seqs_per_worker // seqs_per_chunk
    assert batch % _NUM_WORKERS == 0
    assert seqs_per_worker % seqs_per_chunk == 0
    assert num_chunks % _NBUF == 0 and num_chunks >= 2 * _NBUF
    assert chunk % 8 == 0

    mesh = plsc.VectorSubcoreMesh(
        core_axis_name="c",
        subcore_axis_name="s",
        num_cores=_NUM_CORES,
        num_subcores=_NUM_SUBCORES,
    )

    @functools.partial(
        pl.kernel,
        mesh=mesh,
        out_type=jax.ShapeDtypeStruct((batch, seq, dim), jnp.float32),
        scratch_types=[
            pltpu.VMEM((rows_per_worker,), jnp.int32),
            [pltpu.VMEM((chunk, dim), jnp.float32) for _ in range(_NBUF)],
            [pltpu.SemaphoreType.DMA for _ in range(_NBUF)],
            [pltpu.SemaphoreType.DMA for _ in range(_NBUF)],
        ],
        compiler_params=pltpu.CompilerParams(use_tc_tiling_on_sc=True),
    )
    def gather_kernel(table_hbm, idx_hbm, out_hbm, idx_v, bufs, gsems, ssems):
        wid = lax.axis_index("s") * _NUM_CORES + lax.axis_index("c")
        base = wid * rows_per_worker
        seq_base = wid * seqs_per_worker
        pltpu.sync_copy(idx_hbm.at[pl.ds(base, rows_per_worker)], idx_v)

        def start_gather(chunk_id, b):
            off = chunk_id * chunk
            pltpu.async_copy(
                table_hbm.at[idx_v.at[pl.ds(off, chunk)]], bufs[b], gsems[b]
            )

        def wait_gather(b):
            pltpu.make_async_copy(
                table_hbm.at[pl.ds(0, chunk)], bufs[b], gsems[b]
            ).wait()

        def start_store(chunk_id, b):
            s0 = seq_base + chunk_id * seqs_per_chunk
            for j in range(seqs_per_chunk):
                pltpu.async_copy(
                    bufs[b].at[pl.ds(j * seq, seq)], out_hbm.at[s0 + j], ssems[b]
                )

        def wait_store(b):
            for j in range(seqs_per_chunk):
                pltpu.make_async_copy(
                    bufs[b].at[pl.ds(j * seq, seq)], out_hbm.at[0], ssems[b]
                ).wait()

        start_gather(0, 0)
        start_gather(1, 1)
        wait_gather(0)
        start_store(0, 0)
        start_gather(2, 2)
        wait_gather(1)
        start_store(1, 1)
        start_gather(3, 3)

        def body(g, carry):
            for b in range(_NBUF):
                i = g * _NBUF + b
                wait_store(b)
                start_gather(i, b)
                b2 = (b + 2) % _NBUF
                wait_gather(b2)
                start_store(i - 2, b2)
            return carry

        lax.fori_loop(1, num_chunks // _NBUF, body, 0, unroll=False)

        n = num_chunks
        wait_gather((n - 2) % _NBUF)
        start_store(n - 2, (n - 2) % _NBUF)
        wait_gather((n - 1) % _NBUF)
        start_store(n - 1, (n - 1) % _NBUF)
        for b in range(_NBUF):
            wait_store(b)

    return gather_kernel


def kernel(token_ids, weight):
    dim = weight.shape[1]
    b, s = token_ids.shape
    idx = token_ids.reshape(-1).astype(jnp.int32)
    gather = _make_gather_kernel(b, s, dim, 4)
    return gather(weight, idx)


# speedup vs baseline: 13.9327x; 1.7837x over previous
import functools

import jax
import jax.numpy as jnp
from jax import lax
from jax.experimental import pallas as pl
from jax.experimental.pallas import tpu as pltpu
from jax.experimental.pallas import tpu_sc as plsc

_NUM_CORES = 2
_NUM_SUBCORES = 16
_NUM_WORKERS = _NUM_CORES * _NUM_SUBCORES
_NBUF = 4


@functools.lru_cache(maxsize=None)
def _make_gather_kernel(num_rows: int, dim: int, chunk: int):
    rows_per_worker = num_rows // _NUM_WORKERS
    num_chunks = rows_per_worker // chunk
    assert num_rows % _NUM_WORKERS == 0
    assert rows_per_worker % chunk == 0
    assert num_chunks % _NBUF == 0 and num_chunks >= 2 * _NBUF
    assert chunk % 8 == 0

    mesh = plsc.VectorSubcoreMesh(
        core_axis_name="c",
        subcore_axis_name="s",
        num_cores=_NUM_CORES,
        num_subcores=_NUM_SUBCORES,
    )

    @functools.partial(
        pl.kernel,
        mesh=mesh,
        out_type=jax.ShapeDtypeStruct((num_rows, dim), jnp.float32),
        scratch_types=[
            pltpu.VMEM((rows_per_worker,), jnp.int32),
            [pltpu.VMEM((chunk, dim), jnp.float32) for _ in range(_NBUF)],
            [pltpu.SemaphoreType.DMA for _ in range(_NBUF)],
            [pltpu.SemaphoreType.DMA for _ in range(_NBUF)],
        ],
    )
    def gather_kernel(table_hbm, idx_hbm, out_hbm, idx_v, bufs, gsems, ssems):
        wid = lax.axis_index("s") * _NUM_CORES + lax.axis_index("c")
        base = wid * rows_per_worker
        pltpu.sync_copy(idx_hbm.at[pl.ds(base, rows_per_worker)], idx_v)

        def start_gather(chunk_id, b):
            off = chunk_id * chunk
            pltpu.async_copy(
                table_hbm.at[idx_v.at[pl.ds(off, chunk)]], bufs[b], gsems[b]
            )

        def wait_gather(b):
            pltpu.make_async_copy(
                table_hbm.at[pl.ds(0, chunk)], bufs[b], gsems[b]
            ).wait()

        def start_store(chunk_id, b):
            pltpu.async_copy(
                bufs[b], out_hbm.at[pl.ds(base + chunk_id * chunk, chunk)], ssems[b]
            )

        def wait_store(b):
            pltpu.make_async_copy(
                bufs[b], out_hbm.at[pl.ds(base, chunk)], ssems[b]
            ).wait()

        start_gather(0, 0)
        start_gather(1, 1)
        wait_gather(0)
        start_store(0, 0)
        start_gather(2, 2)
        wait_gather(1)
        start_store(1, 1)
        start_gather(3, 3)

        def body(g, carry):
            for b in range(_NBUF):
                i = g * _NBUF + b
                wait_store(b)
                start_gather(i, b)
                b2 = (b + 2) % _NBUF
                wait_gather(b2)
                start_store(i - 2, b2)
            return carry

        lax.fori_loop(1, num_chunks // _NBUF, body, 0, unroll=False)

        n = num_chunks
        wait_gather((n - 2) % _NBUF)
        start_store(n - 2, (n - 2) % _NBUF)
        wait_gather((n - 1) % _NBUF)
        start_store(n - 1, (n - 1) % _NBUF)
        for b in range(_NBUF):
            wait_store(b)

    return gather_kernel


def kernel(token_ids, weight):
    dim = weight.shape[1]
    b, s = token_ids.shape
    idx = token_ids.T.reshape(-1).astype(jnp.int32)
    gather = _make_gather_kernel(idx.shape[0], dim, 200)
    out = gather(weight, idx)
    return out.reshape(s, b, dim).transpose(1, 0, 2)


# speedup vs baseline: 13.9866x; 1.0039x over previous
import functools

import jax
import jax.numpy as jnp
from jax import lax
from jax.experimental import pallas as pl
from jax.experimental.pallas import tpu as pltpu
from jax.experimental.pallas import tpu_sc as plsc

_NUM_CORES = 2
_NUM_SUBCORES = 16
_NUM_WORKERS = _NUM_CORES * _NUM_SUBCORES


@functools.lru_cache(maxsize=None)
def _make_gather_kernel(num_rows: int, dim: int, chunk: int, nbuf: int):
    rows_per_worker = num_rows // _NUM_WORKERS
    num_chunks = rows_per_worker // chunk
    look = nbuf // 2
    assert num_rows % _NUM_WORKERS == 0
    assert rows_per_worker % chunk == 0
    assert num_chunks % nbuf == 0 and num_chunks >= 2 * nbuf
    assert chunk % 8 == 0

    mesh = plsc.VectorSubcoreMesh(
        core_axis_name="c",
        subcore_axis_name="s",
        num_cores=_NUM_CORES,
        num_subcores=_NUM_SUBCORES,
    )

    @functools.partial(
        pl.kernel,
        mesh=mesh,
        out_type=jax.ShapeDtypeStruct((num_rows, dim), jnp.float32),
        scratch_types=[
            pltpu.VMEM((rows_per_worker,), jnp.int32),
            [pltpu.VMEM((chunk, dim), jnp.float32) for _ in range(nbuf)],
            [pltpu.SemaphoreType.DMA for _ in range(nbuf)],
            [pltpu.SemaphoreType.DMA for _ in range(nbuf)],
        ],
    )
    def gather_kernel(table_hbm, idx_hbm, out_hbm, idx_v, bufs, gsems, ssems):
        wid = lax.axis_index("s") * _NUM_CORES + lax.axis_index("c")
        base = wid * rows_per_worker
        pltpu.sync_copy(idx_hbm.at[pl.ds(base, rows_per_worker)], idx_v)

        def start_gather(chunk_id, b):
            off = chunk_id * chunk
            pltpu.async_copy(
                table_hbm.at[idx_v.at[pl.ds(off, chunk)]], bufs[b], gsems[b]
            )

        def wait_gather(b):
            pltpu.make_async_copy(
                table_hbm.at[pl.ds(0, chunk)], bufs[b], gsems[b]
            ).wait()

        def start_store(chunk_id, b):
            pltpu.async_copy(
                bufs[b], out_hbm.at[pl.ds(base + chunk_id * chunk, chunk)], ssems[b]
            )

        def wait_store(b):
            pltpu.make_async_copy(
                bufs[b], out_hbm.at[pl.ds(base, chunk)], ssems[b]
            ).wait()

        for i in range(look):
            start_gather(i, i)
        for i in range(look, nbuf):
            wait_gather(i - look)
            start_store(i - look, i - look)
            start_gather(i, i)

        def body(g, carry):
            for b in range(nbuf):
                i = g * nbuf + b
                wait_store(b)
                start_gather(i, b)
                b2 = (b + nbuf - look) % nbuf
                wait_gather(b2)
                start_store(i - look, b2)
            return carry

        lax.fori_loop(1, num_chunks // nbuf, body, 0, unroll=False)

        n = num_chunks
        for j in range(n - look, n):
            wait_gather(j % nbuf)
            start_store(j, j % nbuf)
        for b in range(nbuf):
            wait_store(b)

    return gather_kernel


def kernel(token_ids, weight):
    dim = weight.shape[1]
    b, s = token_ids.shape
    idx = token_ids.T.reshape(-1).astype(jnp.int32)
    gather = _make_gather_kernel(idx.shape[0], dim, 80, 8)
    out = gather(weight, idx)
    return out.reshape(s, b, dim).transpose(1, 0, 2)
